# baseline (speedup 1.0000x reference)
import functools
import math

import jax
import jax.numpy as jnp
from jax import lax
from jax.experimental import pallas as pl
from jax.experimental.pallas import tpu as pltpu
from jax.experimental.pallas import tpu_sc as plsc

EPS = 1e-9
N = 10000
D = 128
E = 320000
SCALE = 1.0 / math.sqrt(128.0)

NC = 2
NS = 16
NW = NC * NS
EPT = E // NW
CH = 80
NCHUNK = EPT // CH
NPAD = 10240
RPT = NPAD // NS
ZCH = 128


_GDN = lax.GatherDimensionNumbers(offset_dims=(), collapsed_slice_dims=(0,),
                                  start_index_map=(0,))


def _shuffle(p, idx):
    return lax.gather(p, idx[:, None], _GDN, (1,),
                      mode=lax.GatherScatterMode.PROMISE_IN_BOUNDS)


def _lanesum(p, lane):
    for sh in (8, 4, 2, 1):
        p = p + _shuffle(p, lane ^ sh)
    return p


def _mink_sign(shape):
    col = lax.broadcasted_iota(jnp.int32, shape, 1)
    return jnp.where(col == D - 1, -1.0, 1.0).astype(jnp.float32)


def _row_normalize(a):
    at = a[:, D - 1:D]
    ss = jnp.maximum(jnp.sum(a * a, axis=1, keepdims=True) - at * at, 0.0)
    inv = 1.0 / jnp.maximum(jnp.sqrt(ss), EPS)
    col = lax.broadcasted_iota(jnp.int32, a.shape, 1)
    return jnp.where(col == D - 1, a, a * inv)


def _prep_body(x_ref, wq_ref, bq_ref, wk_ref, bk_ref, wv_ref, bv_ref,
               qn_ref, knm_ref, val_ref, cr_ref):
    x = x_ref[...]
    sgn = _mink_sign((1, D))
    a, b, c, d = x[0:1], x[1:2], x[2:3], x[3:4]
    ac = jnp.sum(a * c * sgn)
    bd = jnp.sum(b * d * sgn)
    ad = jnp.sum(a * d * sgn)
    bc = jnp.sum(b * c * sgn)
    cr_ref[...] = jnp.reshape((ac * bd) / (ad * bc + EPS), (1, 1))

    xp = _row_normalize(x)
    q = jnp.dot(xp, wq_ref[...], preferred_element_type=jnp.float32) + bq_ref[...]
    k = jnp.dot(xp, wk_ref[...], preferred_element_type=jnp.float32) + bk_ref[...]
    v = jnp.dot(xp, wv_ref[...], preferred_element_type=jnp.float32) + bv_ref[...]
    qn_ref[...] = _row_normalize(q)
    kn = _row_normalize(k)
    col = lax.broadcasted_iota(jnp.int32, kn.shape, 1)
    knm_ref[...] = jnp.where(col == D - 1, -kn, kn) * SCALE
    val_ref[...] = v


@functools.partial(jax.jit, static_argnums=())
def _prep(x, Wq, bq, Wk, bk, Wv, bv):
    return pl.pallas_call(
        _prep_body,
        out_shape=[
            jax.ShapeDtypeStruct((N, D), jnp.float32),
            jax.ShapeDtypeStruct((N, D), jnp.float32),
            jax.ShapeDtypeStruct((N, D), jnp.float32),
            jax.ShapeDtypeStruct((1, 1), jnp.float32),
        ],
    )(x, Wq, bq, Wk, bk, Wv, bv)


def _edge_kernel(qn_hbm, knm_hbm, val_hbm, rows_hbm, cols_hbm,
                 acc_hbm, sums_hbm,
                 acc_sp, ridx, cidx, qbuf, kbuf, vbuf, zbuf, sbuf,
                 sem0, sem1, sem2):
    cid = lax.axis_index("c")
    sid = lax.axis_index("s")
    wid = cid * NS + sid

    zrow = jnp.zeros((16,), jnp.float32)

    def zb(i, carry):
        for j in range(D // 16):
            zbuf[i, pl.ds(j * 16, 16)] = zrow
        return carry

    lax.fori_loop(0, ZCH, zb, 0)
    for t in range(RPT // ZCH):
        pltpu.sync_copy(zbuf, acc_sp.at[pl.ds(sid * RPT + t * ZCH, ZCH)])
    plsc.subcore_barrier()

    lane = lax.iota(jnp.int32, 16)

    def chunk(g, lsum):
        ebase = wid * EPT + g * CH
        pltpu.sync_copy(rows_hbm.at[pl.ds(ebase, CH)], ridx)
        pltpu.sync_copy(cols_hbm.at[pl.ds(ebase, CH)], cidx)
        cp0 = pltpu.async_copy(qn_hbm.at[ridx], qbuf, sem0)
        cp1 = pltpu.async_copy(knm_hbm.at[cidx], kbuf, sem1)
        cp2 = pltpu.async_copy(val_hbm.at[cidx], vbuf, sem2)
        cp0.wait()
        cp1.wait()
        cp2.wait()

        def edot(e, ls):
            p = qbuf[e, pl.ds(0, 16)] * kbuf[e, pl.ds(0, 16)]
            for j in range(1, D // 16):
                p = p + qbuf[e, pl.ds(j * 16, 16)] * kbuf[e, pl.ds(j * 16, 16)]
            w = jnp.exp(_lanesum(p, lane))
            for j in range(D // 16):
                vbuf[e, pl.ds(j * 16, 16)] = vbuf[e, pl.ds(j * 16, 16)] * w
            return ls + w

        lsum = plsc.parallel_loop(0, CH, 1, unroll=4, carry=lsum)(edot)
        pltpu.sync_copy(vbuf, acc_sp.at[ridx], add=True)
        return lsum

    lsum = lax.fori_loop(0, NCHUNK, chunk, jnp.zeros((16,), jnp.float32))

    sbuf[:] = lsum
    pltpu.sync_copy(sbuf, sums_hbm.at[pl.ds(wid * 16, 16)])

    plsc.subcore_barrier()
    for t in range(RPT // ZCH):
        sl = pl.ds(sid * RPT + t * ZCH, ZCH)
        pltpu.sync_copy(acc_sp.at[sl], acc_hbm.at[cid, sl])


def _edge(qn, knm, vals, rows, cols):
    mesh = plsc.VectorSubcoreMesh(core_axis_name="c", subcore_axis_name="s")
    f = functools.partial(
        pl.kernel,
        mesh=mesh,
        out_type=[
            jax.ShapeDtypeStruct((NC, NPAD, D), jnp.float32),
            jax.ShapeDtypeStruct((NW * 16,), jnp.float32),
        ],
        scratch_types=[
            pltpu.VMEM_SHARED((NPAD, D), jnp.float32),
            pltpu.VMEM((CH,), jnp.int32),
            pltpu.VMEM((CH,), jnp.int32),
            pltpu.VMEM((CH, D), jnp.float32),
            pltpu.VMEM((CH, D), jnp.float32),
            pltpu.VMEM((CH, D), jnp.float32),
            pltpu.VMEM((ZCH, D), jnp.float32),
            pltpu.VMEM((16,), jnp.float32),
            pltpu.SemaphoreType.DMA,
            pltpu.SemaphoreType.DMA,
            pltpu.SemaphoreType.DMA,
        ],
    )(_edge_kernel)
    return f(qn, knm, vals, rows, cols)


def _fin_body(acc_ref, sums_ref, wo_ref, bo_ref, cr_ref, out_ref):
    A = acc_ref[0, 0:N, :] + acc_ref[1, 0:N, :]
    Z = jnp.sum(sums_ref[...][:, 0:1])
    o = (jnp.dot(A, wo_ref[...], preferred_element_type=jnp.float32) * (1.0 / Z)
         + bo_ref[...])
    sgn = _mink_sign((1, D))
    a, b, c, d = o[0:1], o[1:2], o[2:3], o[3:4]
    ac = jnp.sum(a * c * sgn)
    bd = jnp.sum(b * d * sgn)
    ad = jnp.sum(a * d * sgn)
    bc = jnp.sum(b * c * sgn)
    cr_now = (ac * bd) / (ad * bc + EPS)
    tgt = cr_ref[0, 0]
    scale = jnp.where(jnp.abs(cr_now) > EPS,
                      jnp.sqrt(jnp.abs(tgt / (cr_now + EPS))),
                      1.0)
    out_ref[...] = o * scale


def _finish(acc, sums, Wo, bo, cr):
    return pl.pallas_call(
        _fin_body,
        out_shape=jax.ShapeDtypeStruct((N, D), jnp.float32),
    )(acc, sums, Wo, bo, cr)


def kernel(x, edge_index, Wq, bq, Wk, bk, Wv, bv, Wo, bo):
    rows = edge_index[0].astype(jnp.int32)
    cols = edge_index[1].astype(jnp.int32)
    qn, knm, vals, cr = _prep(x, Wq, bq.reshape(1, D), Wk, bk.reshape(1, D),
                              Wv, bv.reshape(1, D))
    acc, sums = _edge(qn, knm, vals, rows, cols)
    return _finish(acc, sums.reshape(NW, 16), Wo, bo.reshape(1, D), cr)


# speedup vs baseline: 1.2211x; 1.2211x over previous
import functools
import math

import jax
import jax.numpy as jnp
from jax import lax
from jax.experimental import pallas as pl
from jax.experimental.pallas import tpu as pltpu
from jax.experimental.pallas import tpu_sc as plsc

EPS = 1e-9
N = 10000
D = 128
E = 320000
SCALE = 1.0 / math.sqrt(128.0)

NC = 2
NS = 16
NW = NC * NS
EPT = E // NW
CH = 40
NCHUNK = EPT // CH
NCOMP = 252
PADC = 256
NPAD = 10240
RPT = NPAD // NS


_GDN = lax.GatherDimensionNumbers(offset_dims=(), collapsed_slice_dims=(0,),
                                  start_index_map=(0,))


def _shuffle(p, idx):
    return lax.gather(p, idx[:, None], _GDN, (1,),
                      mode=lax.GatherScatterMode.PROMISE_IN_BOUNDS)


def _lanesum(p, lane):
    for sh in (8, 4, 2, 1):
        p = p + _shuffle(p, lane ^ sh)
    return p


def _mink_sign(shape):
    col = lax.broadcasted_iota(jnp.int32, shape, 1)
    return jnp.where(col == D - 1, -1.0, 1.0).astype(jnp.float32)


def _row_normalize(a):
    at = a[:, D - 1:D]
    ss = jnp.maximum(jnp.sum(a * a, axis=1, keepdims=True) - at * at, 0.0)
    inv = 1.0 / jnp.maximum(jnp.sqrt(ss), EPS)
    col = lax.broadcasted_iota(jnp.int32, a.shape, 1)
    return jnp.where(col == D - 1, a, a * inv)


def _prep_body(x_ref, wq_ref, bq_ref, wk_ref, bk_ref, wv_ref, bv_ref,
               qn_ref, knm_ref, val_ref, cr_ref):
    x = x_ref[...]
    sgn = _mink_sign((1, D))
    a, b, c, d = x[0:1], x[1:2], x[2:3], x[3:4]
    ac = jnp.sum(a * c * sgn)
    bd = jnp.sum(b * d * sgn)
    ad = jnp.sum(a * d * sgn)
    bc = jnp.sum(b * c * sgn)
    cr_ref[...] = jnp.reshape((ac * bd) / (ad * bc + EPS), (1, 1))

    xp = _row_normalize(x)
    q = jnp.dot(xp, wq_ref[...], preferred_element_type=jnp.float32) + bq_ref[...]
    k = jnp.dot(xp, wk_ref[...], preferred_element_type=jnp.float32) + bk_ref[...]
    v = jnp.dot(xp, wv_ref[...], preferred_element_type=jnp.float32) + bv_ref[...]
    qn_ref[...] = _row_normalize(q)
    kn = _row_normalize(k)
    col = lax.broadcasted_iota(jnp.int32, kn.shape, 1)
    knm_ref[...] = jnp.where(col == D - 1, -kn, kn) * SCALE
    val_ref[...] = v


@functools.partial(jax.jit, static_argnums=())
def _prep(x, Wq, bq, Wk, bk, Wv, bv):
    return pl.pallas_call(
        _prep_body,
        out_shape=[
            jax.ShapeDtypeStruct((N, D), jnp.float32),
            jax.ShapeDtypeStruct((N, D), jnp.float32),
            jax.ShapeDtypeStruct((N, D), jnp.float32),
            jax.ShapeDtypeStruct((1, 1), jnp.float32),
        ],
    )(x, Wq, bq, Wk, bk, Wv, bv)


def _edge_kernel(qn_hbm, knm_hbm, val_hbm, rows_hbm, cols_hbm,
                 acc_hbm, sums_hbm,
                 acc_sp,
                 ri0, ri1, ri2, ri3, ci0, ci1, ci2, ci3,
                 qb0, kb0, vb0, qb1, kb1, vb1, sbuf,
                 sr0, sr1, sr2, sr3, sc0, sc1, sc2, sc3,
                 sq0, sk0, sv0, sq1, sk1, sv1):
    cid = lax.axis_index("c")
    sid = lax.axis_index("s")
    wid = cid * NS + sid
    ridxs, cidxs = (ri0, ri1, ri2, ri3), (ci0, ci1, ci2, ci3)
    rsems = (sr0, sr1, sr2, sr3)
    csems = (sc0, sc1, sc2, sc3)
    qbufs, kbufs, vbufs = (qb0, qb1), (kb0, kb1), (vb0, vb1)
    dsems = ((sq0, sk0, sv0), (sq1, sk1, sv1))

    zrow = jnp.zeros((16,), jnp.float32)

    def zb(i, carry):
        for j in range(D // 16):
            qb0[i, pl.ds(j * 16, 16)] = zrow
        return carry

    lax.fori_loop(0, CH, zb, 0)
    for t in range(RPT // CH):
        pltpu.sync_copy(qb0, acc_sp.at[pl.ds(sid * RPT + t * CH, CH)])
    plsc.subcore_barrier()

    lane = lax.iota(jnp.int32, 16)

    def issue_idx(g, b4):
        pltpu.async_copy(rows_hbm.at[wid, g], ridxs[b4], rsems[b4])
        pltpu.async_copy(cols_hbm.at[wid, g], cidxs[b4], csems[b4])

    def wait_idx(g, b4):
        pltpu.make_async_copy(rows_hbm.at[wid, g], ridxs[b4], rsems[b4]).wait()
        pltpu.make_async_copy(cols_hbm.at[wid, g], cidxs[b4], csems[b4]).wait()

    def issue_data(g, b2, b4):
        pltpu.async_copy(qn_hbm.at[ridxs[b4]], qbufs[b2], dsems[b2][0])
        pltpu.async_copy(knm_hbm.at[cidxs[b4]], kbufs[b2], dsems[b2][1])
        pltpu.async_copy(val_hbm.at[cidxs[b4]], vbufs[b2], dsems[b2][2])

    def wait_data(g, b2, b4):
        pltpu.make_async_copy(qn_hbm.at[ridxs[b4]], qbufs[b2], dsems[b2][0]).wait()
        pltpu.make_async_copy(knm_hbm.at[cidxs[b4]], kbufs[b2], dsems[b2][1]).wait()
        pltpu.make_async_copy(val_hbm.at[cidxs[b4]], vbufs[b2], dsems[b2][2]).wait()

    def compute(g, b2, b4, valid, lsum):
        qbuf, kbuf, vbuf = qbufs[b2], kbufs[b2], vbufs[b2]

        def edot(e, ls):
            p = qbuf[e, pl.ds(0, 16)] * kbuf[e, pl.ds(0, 16)]
            for j in range(1, D // 16):
                p = p + qbuf[e, pl.ds(j * 16, 16)] * kbuf[e, pl.ds(j * 16, 16)]
            w = jnp.exp(_lanesum(p, lane))
            w = jnp.where(valid, w, 0.0)
            for j in range(D // 16):
                vbuf[e, pl.ds(j * 16, 16)] = vbuf[e, pl.ds(j * 16, 16)] * w
            return ls + w

        lsum = plsc.parallel_loop(0, CH, 1, unroll=4, carry=lsum)(edot)
        pltpu.sync_copy(vbuf, acc_sp.at[ridxs[b4]], add=True)
        return lsum

    issue_idx(0, 0)
    issue_idx(1, 1)
    issue_idx(2, 2)
    wait_idx(0, 0)
    issue_data(0, 0, 0)

    def quad(go, lsum):
        g0 = go * 4
        for c in range(4):
            g = g0 + c
            b2, b4 = c % 2, c
            nb2, nb4 = (c + 1) % 2, (c + 1) % 4
            wait_idx(g + 1, nb4)
            issue_data(g + 1, nb2, nb4)
            wait_data(g, b2, b4)
            lsum = compute(g, b2, b4, g < NCHUNK, lsum)
            issue_idx(g + 3, (c + 3) % 4)
        return lsum

    lsum = lax.fori_loop(0, NCOMP // 4, quad, jnp.zeros((16,), jnp.float32))

    wait_idx(NCOMP + 1, (NCOMP + 1) % 4)
    wait_idx(NCOMP + 2, (NCOMP + 2) % 4)
    wait_data(NCOMP, NCOMP % 2, NCOMP % 4)

    sbuf[:] = lsum
    pltpu.sync_copy(sbuf, sums_hbm.at[pl.ds(wid * 16, 16)])

    plsc.subcore_barrier()
    for t in range(RPT // CH):
        sl = pl.ds(sid * RPT + t * CH, CH)
        pltpu.sync_copy(acc_sp.at[sl], acc_hbm.at[cid, sl])


def _edge(qn, knm, vals, rows, cols):
    mesh = plsc.VectorSubcoreMesh(core_axis_name="c", subcore_axis_name="s")
    f = functools.partial(
        pl.kernel,
        mesh=mesh,
        out_type=[
            jax.ShapeDtypeStruct((NC, NPAD, D), jnp.float32),
            jax.ShapeDtypeStruct((NW * 16,), jnp.float32),
        ],
        scratch_types=(
            [pltpu.VMEM_SHARED((NPAD, D), jnp.float32)]
            + [pltpu.VMEM((CH,), jnp.int32)] * 8
            + [pltpu.VMEM((CH, D), jnp.float32)] * 6
            + [pltpu.VMEM((16,), jnp.float32)]
            + [pltpu.SemaphoreType.DMA] * 14
        ),
    )(_edge_kernel)
    pad = ((0, 0), (0, PADC - NCHUNK), (0, 0))
    return f(qn, knm, vals,
             jnp.pad(rows.reshape(NW, NCHUNK, CH), pad),
             jnp.pad(cols.reshape(NW, NCHUNK, CH), pad))


def _fin_body(acc_ref, sums_ref, wo_ref, bo_ref, cr_ref, out_ref):
    A = acc_ref[0, 0:N, :] + acc_ref[1, 0:N, :]
    Z = jnp.sum(sums_ref[...][:, 0:1])
    o = (jnp.dot(A, wo_ref[...], preferred_element_type=jnp.float32) * (1.0 / Z)
         + bo_ref[...])
    sgn = _mink_sign((1, D))
    a, b, c, d = o[0:1], o[1:2], o[2:3], o[3:4]
    ac = jnp.sum(a * c * sgn)
    bd = jnp.sum(b * d * sgn)
    ad = jnp.sum(a * d * sgn)
    bc = jnp.sum(b * c * sgn)
    cr_now = (ac * bd) / (ad * bc + EPS)
    tgt = cr_ref[0, 0]
    scale = jnp.where(jnp.abs(cr_now) > EPS,
                      jnp.sqrt(jnp.abs(tgt / (cr_now + EPS))),
                      1.0)
    out_ref[...] = o * scale


def _finish(acc, sums, Wo, bo, cr):
    return pl.pallas_call(
        _fin_body,
        out_shape=jax.ShapeDtypeStruct((N, D), jnp.float32),
    )(acc, sums, Wo, bo, cr)


def kernel(x, edge_index, Wq, bq, Wk, bk, Wv, bv, Wo, bo):
    rows = edge_index[0].astype(jnp.int32)
    cols = edge_index[1].astype(jnp.int32)
    qn, knm, vals, cr = _prep(x, Wq, bq.reshape(1, D), Wk, bk.reshape(1, D),
                              Wv, bv.reshape(1, D))
    acc, sums = _edge(qn, knm, vals, rows, cols)
    return _finish(acc, sums.reshape(NW, 16), Wo, bo.reshape(1, D), cr)


# speedup vs baseline: 1.2785x; 1.0470x over previous
import functools
import math

import jax
import jax.numpy as jnp
from jax import lax
from jax.experimental import pallas as pl
from jax.experimental.pallas import tpu as pltpu
from jax.experimental.pallas import tpu_sc as plsc

EPS = 1e-9
N = 10000
D = 128
E = 320000
SCALE = 1.0 / math.sqrt(128.0)

NC = 2
NS = 16
NW = NC * NS
EPT = E // NW
CH = 40
NCHUNK = EPT // CH
NCOMP = 252
PADC = 256
NPAD = 10240
RPT = NPAD // NS


_GDN = lax.GatherDimensionNumbers(offset_dims=(), collapsed_slice_dims=(0,),
                                  start_index_map=(0,))


def _shuffle(p, idx):
    return lax.gather(p, idx[:, None], _GDN, (1,),
                      mode=lax.GatherScatterMode.PROMISE_IN_BOUNDS)


def _lanesum(p, lane):
    for sh in (8, 4, 2, 1):
        p = p + _shuffle(p, lane ^ sh)
    return p


def _mink_sign(shape):
    col = lax.broadcasted_iota(jnp.int32, shape, 1)
    return jnp.where(col == D - 1, -1.0, 1.0).astype(jnp.float32)


def _row_normalize(a):
    at = a[:, D - 1:D]
    ss = jnp.maximum(jnp.sum(a * a, axis=1, keepdims=True) - at * at, 0.0)
    inv = 1.0 / jnp.maximum(jnp.sqrt(ss), EPS)
    col = lax.broadcasted_iota(jnp.int32, a.shape, 1)
    return jnp.where(col == D - 1, a, a * inv)


def _prep_body(x_ref, wq_ref, bq_ref, wk_ref, bk_ref, wv_ref, bv_ref,
               qn_ref, knm_ref, val_ref, cr_ref):
    x = x_ref[...]
    sgn = _mink_sign((1, D))
    a, b, c, d = x[0:1], x[1:2], x[2:3], x[3:4]
    ac = jnp.sum(a * c * sgn)
    bd = jnp.sum(b * d * sgn)
    ad = jnp.sum(a * d * sgn)
    bc = jnp.sum(b * c * sgn)
    cr_ref[...] = jnp.reshape((ac * bd) / (ad * bc + EPS), (1, 1))

    xp = _row_normalize(x)
    q = jnp.dot(xp, wq_ref[...], preferred_element_type=jnp.float32) + bq_ref[...]
    k = jnp.dot(xp, wk_ref[...], preferred_element_type=jnp.float32) + bk_ref[...]
    v = jnp.dot(xp, wv_ref[...], preferred_element_type=jnp.float32) + bv_ref[...]
    qn_ref[...] = _row_normalize(q)
    kn = _row_normalize(k)
    col = lax.broadcasted_iota(jnp.int32, kn.shape, 1)
    knm_ref[...] = jnp.where(col == D - 1, -kn, kn) * SCALE
    val_ref[...] = v


@functools.partial(jax.jit, static_argnums=())
def _prep(x, Wq, bq, Wk, bk, Wv, bv):
    return pl.pallas_call(
        _prep_body,
        out_shape=[
            jax.ShapeDtypeStruct((N, D), jnp.float32),
            jax.ShapeDtypeStruct((N, D), jnp.float32),
            jax.ShapeDtypeStruct((N, D), jnp.float32),
            jax.ShapeDtypeStruct((1, 1), jnp.float32),
        ],
    )(x, Wq, bq, Wk, bk, Wv, bv)


def _edge_kernel(qn_hbm, knm_hbm, val_hbm, rows_hbm, cols_hbm,
                 acc_hbm, sums_hbm,
                 acc_sp,
                 ri0, ri1, ri2, ri3, ci0, ci1, ci2, ci3,
                 qb0, kb0, vb0, qb1, kb1, vb1, sbuf,
                 sr0, sr1, sr2, sr3, sc0, sc1, sc2, sc3,
                 sq0, sk0, sv0, sq1, sk1, sv1, ss0, ss1):
    cid = lax.axis_index("c")
    sid = lax.axis_index("s")
    wid = cid * NS + sid
    ridxs, cidxs = (ri0, ri1, ri2, ri3), (ci0, ci1, ci2, ci3)
    rsems = (sr0, sr1, sr2, sr3)
    csems = (sc0, sc1, sc2, sc3)
    qbufs, kbufs, vbufs = (qb0, qb1), (kb0, kb1), (vb0, vb1)
    dsems = ((sq0, sk0, sv0), (sq1, sk1, sv1))
    ssems = (ss0, ss1)

    zrow = jnp.zeros((16,), jnp.float32)

    def zb(i, carry):
        for j in range(D // 16):
            qb0[i, pl.ds(j * 16, 16)] = zrow
        return carry

    lax.fori_loop(0, CH, zb, 0)
    for t in range(RPT // CH):
        pltpu.sync_copy(qb0, acc_sp.at[pl.ds(sid * RPT + t * CH, CH)])
    plsc.subcore_barrier()

    lane = lax.iota(jnp.int32, 16)

    def issue_idx(g, b4):
        pltpu.async_copy(rows_hbm.at[wid, g], ridxs[b4], rsems[b4])
        pltpu.async_copy(cols_hbm.at[wid, g], cidxs[b4], csems[b4])

    def wait_idx(g, b4):
        pltpu.make_async_copy(rows_hbm.at[wid, g], ridxs[b4], rsems[b4]).wait()
        pltpu.make_async_copy(cols_hbm.at[wid, g], cidxs[b4], csems[b4]).wait()

    def issue_data(g, b2, b4):
        pltpu.async_copy(qn_hbm.at[ridxs[b4]], qbufs[b2], dsems[b2][0])
        pltpu.async_copy(knm_hbm.at[cidxs[b4]], kbufs[b2], dsems[b2][1])
        pltpu.async_copy(val_hbm.at[cidxs[b4]], vbufs[b2], dsems[b2][2])

    def wait_data(g, b2, b4):
        pltpu.make_async_copy(qn_hbm.at[ridxs[b4]], qbufs[b2], dsems[b2][0]).wait()
        pltpu.make_async_copy(knm_hbm.at[cidxs[b4]], kbufs[b2], dsems[b2][1]).wait()
        pltpu.make_async_copy(val_hbm.at[cidxs[b4]], vbufs[b2], dsems[b2][2]).wait()

    def compute(g, b2, b4, valid, lsum):
        qbuf, kbuf, vbuf = qbufs[b2], kbufs[b2], vbufs[b2]

        def edot(e, ls):
            p = qbuf[e, pl.ds(0, 16)] * kbuf[e, pl.ds(0, 16)]
            for j in range(1, D // 16):
                p = p + qbuf[e, pl.ds(j * 16, 16)] * kbuf[e, pl.ds(j * 16, 16)]
            w = jnp.exp(_lanesum(p, lane))
            w = jnp.where(valid, w, 0.0)
            for j in range(D // 16):
                vbuf[e, pl.ds(j * 16, 16)] = vbuf[e, pl.ds(j * 16, 16)] * w
            return ls + w

        lsum = plsc.parallel_loop(0, CH, 1, unroll=4, carry=lsum)(edot)
        pb2, pb4 = (b2 + 1) % 2, (b4 + 3) % 4

        @pl.when(g >= 1)
        def _():
            pltpu.make_async_copy(vbufs[pb2], acc_sp.at[ridxs[pb4]],
                                  ssems[pb2]).wait()

        pltpu.async_copy(vbuf, acc_sp.at[ridxs[b4]], ssems[b2], add=True)
        return lsum

    issue_idx(0, 0)
    issue_idx(1, 1)
    issue_idx(2, 2)
    wait_idx(0, 0)
    issue_data(0, 0, 0)

    def quad(go, lsum):
        g0 = go * 4
        for c in range(4):
            g = g0 + c
            b2, b4 = c % 2, c
            nb2, nb4 = (c + 1) % 2, (c + 1) % 4
            wait_idx(g + 1, nb4)
            issue_data(g + 1, nb2, nb4)
            wait_data(g, b2, b4)
            lsum = compute(g, b2, b4, g < NCHUNK, lsum)
            issue_idx(g + 3, (c + 3) % 4)
        return lsum

    lsum = lax.fori_loop(0, NCOMP // 4, quad, jnp.zeros((16,), jnp.float32))

    wait_idx(NCOMP + 1, (NCOMP + 1) % 4)
    wait_idx(NCOMP + 2, (NCOMP + 2) % 4)
    wait_data(NCOMP, NCOMP % 2, NCOMP % 4)
    gl = NCOMP - 1
    pltpu.make_async_copy(vbufs[gl % 2], acc_sp.at[ridxs[gl % 4]],
                          ssems[gl % 2]).wait()

    sbuf[:] = lsum
    pltpu.sync_copy(sbuf, sums_hbm.at[pl.ds(wid * 16, 16)])

    plsc.subcore_barrier()
    for t in range(RPT // CH):
        sl = pl.ds(sid * RPT + t * CH, CH)
        pltpu.sync_copy(acc_sp.at[sl], acc_hbm.at[cid, sl])


def _edge(qn, knm, vals, rows, cols):
    mesh = plsc.VectorSubcoreMesh(core_axis_name="c", subcore_axis_name="s")
    f = functools.partial(
        pl.kernel,
        mesh=mesh,
        out_type=[
            jax.ShapeDtypeStruct((NC, NPAD, D), jnp.float32),
            jax.ShapeDtypeStruct((NW * 16,), jnp.float32),
        ],
        scratch_types=(
            [pltpu.VMEM_SHARED((NPAD, D), jnp.float32)]
            + [pltpu.VMEM((CH,), jnp.int32)] * 8
            + [pltpu.VMEM((CH, D), jnp.float32)] * 6
            + [pltpu.VMEM((16,), jnp.float32)]
            + [pltpu.SemaphoreType.DMA] * 16
        ),
    )(_edge_kernel)
    pad = ((0, 0), (0, PADC - NCHUNK), (0, 0))
    return f(qn, knm, vals,
             jnp.pad(rows.reshape(NW, NCHUNK, CH), pad),
             jnp.pad(cols.reshape(NW, NCHUNK, CH), pad))


def _fin_body(acc_ref, sums_ref, wo_ref, bo_ref, cr_ref, out_ref):
    A = acc_ref[0, 0:N, :] + acc_ref[1, 0:N, :]
    Z = jnp.sum(sums_ref[...][:, 0:1])
    o = (jnp.dot(A, wo_ref[...], preferred_element_type=jnp.float32) * (1.0 / Z)
         + bo_ref[...])
    sgn = _mink_sign((1, D))
    a, b, c, d = o[0:1], o[1:2], o[2:3], o[3:4]
    ac = jnp.sum(a * c * sgn)
    bd = jnp.sum(b * d * sgn)
    ad = jnp.sum(a * d * sgn)
    bc = jnp.sum(b * c * sgn)
    cr_now = (ac * bd) / (ad * bc + EPS)
    tgt = cr_ref[0, 0]
    scale = jnp.where(jnp.abs(cr_now) > EPS,
                      jnp.sqrt(jnp.abs(tgt / (cr_now + EPS))),
                      1.0)
    out_ref[...] = o * scale


def _finish(acc, sums, Wo, bo, cr):
    return pl.pallas_call(
        _fin_body,
        out_shape=jax.ShapeDtypeStruct((N, D), jnp.float32),
    )(acc, sums, Wo, bo, cr)


def kernel(x, edge_index, Wq, bq, Wk, bk, Wv, bv, Wo, bo):
    rows = edge_index[0].astype(jnp.int32)
    cols = edge_index[1].astype(jnp.int32)
    qn, knm, vals, cr = _prep(x, Wq, bq.reshape(1, D), Wk, bk.reshape(1, D),
                              Wv, bv.reshape(1, D))
    acc, sums = _edge(qn, knm, vals, rows, cols)
    return _finish(acc, sums.reshape(NW, 16), Wo, bo.reshape(1, D), cr)
